# Initial kernel scaffold; baseline (speedup 1.0000x reference)
import jax
import jax.numpy as jnp
from jax.experimental import pallas as pl


def kernel(x, codebook):
    raise NotImplementedError("write your pallas kernel here")


# speedup vs baseline: 1.6930x; 1.6930x over previous
import jax
import jax.numpy as jnp
from jax.experimental import pallas as pl
from jax.experimental.pallas import tpu as pltpu

_K = 512
_D = 256
_BLK = 1024


def _vq_body(x_ref, cb_ref, quant_ref, idx_ref, loss_ref):
    x = x_ref[...]
    cb = cb_ref[...]
    cross = jax.lax.dot_general(
        x, cb, (((1,), (1,)), ((), ())), preferred_element_type=jnp.float32
    )
    x_sq = jnp.sum(x * x, axis=1, keepdims=True)
    c_sq = jnp.sum(cb * cb, axis=1)
    dist_sq = jnp.maximum(x_sq + c_sq[None, :] - 2.0 * cross, 0.0)
    dists = jnp.sqrt(dist_sq)
    min_d = jnp.min(dists, axis=1, keepdims=True)
    iota_k = jax.lax.broadcasted_iota(jnp.int32, dists.shape, 1)
    idx = jnp.min(jnp.where(dists == min_d, iota_k, _K), axis=1)
    idx_ref[...] = idx.reshape(1, 1, _BLK)
    onehot = (iota_k == idx[:, None]).astype(jnp.float32)
    gathered = jax.lax.dot_general(
        onehot, cb, (((1,), (0,)), ((), ())),
        preferred_element_type=jnp.float32,
        precision=jax.lax.Precision.HIGHEST,
    )
    quant = x + (gathered - x)
    quant_ref[...] = quant
    diff = quant - x

    @pl.when(pl.program_id(0) == 0)
    def _():
        loss_ref[0, 0] = 0.0

    loss_ref[0, 0] += jnp.sum(diff * diff)


def kernel(x, codebook):
    B, T, D = x.shape
    N = B * T
    xf = x.reshape(N, D)
    grid = N // _BLK
    quant, idx, loss_sum = pl.pallas_call(
        _vq_body,
        grid=(grid,),
        in_specs=[
            pl.BlockSpec((_BLK, D), lambda i: (i, 0)),
            pl.BlockSpec((_K, D), lambda i: (0, 0)),
        ],
        out_specs=[
            pl.BlockSpec((_BLK, D), lambda i: (i, 0)),
            pl.BlockSpec((1, 1, _BLK), lambda i: (i, 0, 0)),
            pl.BlockSpec(memory_space=pltpu.SMEM),
        ],
        out_shape=[
            jax.ShapeDtypeStruct((N, D), jnp.float32),
            jax.ShapeDtypeStruct((grid, 1, _BLK), jnp.int32),
            jax.ShapeDtypeStruct((1, 1), jnp.float32),
        ],
    )(xf, codebook)
    quantized = quant.reshape(B, T, D)
    indices = idx.reshape(B, T)
    quantize_loss = (2.0 / N / D) * loss_sum[0, 0]
    return (quantized, indices, quantize_loss)


# speedup vs baseline: 2.6896x; 1.5886x over previous
import jax
import jax.numpy as jnp
from jax.experimental import pallas as pl
from jax.experimental.pallas import tpu as pltpu

_K = 512
_D = 256
_BLK = 1024


def _vq_body(x_ref, cb_ref, quant_ref, idx_ref, loss_ref, csq_ref):
    x = x_ref[...]
    cb = cb_ref[...]

    @pl.when(pl.program_id(0) == 0)
    def _():
        csq_ref[...] = jnp.sum(cb * cb, axis=1, keepdims=True).reshape(1, _K)
        loss_ref[0, 0] = 0.0

    cross = jax.lax.dot_general(
        x, cb, (((1,), (1,)), ((), ())), preferred_element_type=jnp.float32
    )
    x_sq = jnp.sum(x * x, axis=1, keepdims=True)
    c_sq = csq_ref[...]
    dist_sq = jnp.maximum(x_sq + c_sq - 2.0 * cross, 0.0)
    dists = jnp.sqrt(dist_sq)
    min_d = jnp.min(dists, axis=1, keepdims=True)
    iota_f = jax.lax.broadcasted_iota(jnp.int32, dists.shape, 1).astype(jnp.float32)
    idx_f = jnp.min(
        jnp.where(dists == min_d, iota_f, float(_K)), axis=1, keepdims=True
    )
    idx_ref[...] = idx_f.astype(jnp.int32).reshape(1, 1, _BLK)
    onehot = jnp.where(iota_f == idx_f, 1.0, 0.0).astype(jnp.bfloat16)
    gathered = jax.lax.dot_general(
        onehot, cb.astype(jnp.bfloat16), (((1,), (0,)), ((), ())),
        preferred_element_type=jnp.float32,
    )
    quant = x + (gathered - x)
    quant_ref[...] = quant
    diff = quant - x
    loss_ref[0, 0] += jnp.sum(diff * diff)


def kernel(x, codebook):
    B, T, D = x.shape
    N = B * T
    xf = x.reshape(N, D)
    grid = N // _BLK
    quant, idx, loss_sum = pl.pallas_call(
        _vq_body,
        grid=(grid,),
        in_specs=[
            pl.BlockSpec((_BLK, D), lambda i: (i, 0)),
            pl.BlockSpec((_K, D), lambda i: (0, 0)),
        ],
        out_specs=[
            pl.BlockSpec((_BLK, D), lambda i: (i, 0)),
            pl.BlockSpec((1, 1, _BLK), lambda i: (i, 0, 0)),
            pl.BlockSpec(memory_space=pltpu.SMEM),
        ],
        out_shape=[
            jax.ShapeDtypeStruct((N, D), jnp.float32),
            jax.ShapeDtypeStruct((grid, 1, _BLK), jnp.int32),
            jax.ShapeDtypeStruct((1, 1), jnp.float32),
        ],
        scratch_shapes=[pltpu.VMEM((1, _K), jnp.float32)],
    )(xf, codebook)
    quantized = quant.reshape(B, T, D)
    indices = idx.reshape(B, T)
    quantize_loss = (2.0 / N / D) * loss_sum[0, 0]
    return (quantized, indices, quantize_loss)


# speedup vs baseline: 2.7241x; 1.0128x over previous
import jax
import jax.numpy as jnp
from jax.experimental import pallas as pl
from jax.experimental.pallas import tpu as pltpu

_K = 512
_D = 256
_BLK = 1024


def _vq_body(x_ref, cb_ref, quant_ref, idx_ref, loss_ref, csq_ref):
    x = x_ref[...]
    cb = cb_ref[...]

    @pl.when(pl.program_id(0) == 0)
    def _():
        csq_ref[...] = jnp.sum(cb * cb, axis=1, keepdims=True).reshape(1, _K)
        loss_ref[0, 0] = 0.0

    cross = jax.lax.dot_general(
        x, cb, (((1,), (1,)), ((), ())), preferred_element_type=jnp.float32
    )
    x_sq = jnp.sum(x * x, axis=1, keepdims=True)
    c_sq = csq_ref[...]
    dist_sq = jnp.maximum(x_sq + c_sq - 2.0 * cross, 0.0)
    dists = jnp.sqrt(dist_sq)
    min_d = jnp.min(dists, axis=1, keepdims=True)
    iota_f = jax.lax.broadcasted_iota(jnp.int32, dists.shape, 1).astype(jnp.float32)
    idx_f = jnp.min(
        jnp.where(dists == min_d, iota_f, float(_K)), axis=1, keepdims=True
    )
    idx_ref[...] = idx_f.astype(jnp.int32).reshape(1, 1, _BLK)
    onehot = jnp.where(iota_f == idx_f, 1.0, 0.0).astype(jnp.bfloat16)
    gathered = jax.lax.dot_general(
        onehot, cb.astype(jnp.bfloat16), (((1,), (0,)), ((), ())),
        preferred_element_type=jnp.float32,
    )
    quant_ref[...] = gathered
    loss_ref[0, 0] += jnp.sum(min_d * min_d)


def kernel(x, codebook):
    B, T, D = x.shape
    N = B * T
    xf = x.reshape(N, D)
    grid = N // _BLK
    quant, idx, loss_sum = pl.pallas_call(
        _vq_body,
        grid=(grid,),
        in_specs=[
            pl.BlockSpec((_BLK, D), lambda i: (i, 0)),
            pl.BlockSpec((_K, D), lambda i: (0, 0)),
        ],
        out_specs=[
            pl.BlockSpec((_BLK, D), lambda i: (i, 0)),
            pl.BlockSpec((1, 1, _BLK), lambda i: (i, 0, 0)),
            pl.BlockSpec(memory_space=pltpu.SMEM),
        ],
        out_shape=[
            jax.ShapeDtypeStruct((N, D), jnp.float32),
            jax.ShapeDtypeStruct((grid, 1, _BLK), jnp.int32),
            jax.ShapeDtypeStruct((1, 1), jnp.float32),
        ],
        scratch_shapes=[pltpu.VMEM((1, _K), jnp.float32)],
    )(xf, codebook)
    quantized = quant.reshape(B, T, D)
    indices = idx.reshape(B, T)
    quantize_loss = (2.0 / N / D) * loss_sum[0, 0]
    return (quantized, indices, quantize_loss)


# speedup vs baseline: 2.8316x; 1.0395x over previous
import jax
import jax.numpy as jnp
from jax.experimental import pallas as pl
from jax.experimental.pallas import tpu as pltpu

_K = 512
_D = 256
_BLK = 2048


def _vq_body(x_ref, cb_ref, quant_ref, idx_ref, loss_ref, csq_ref):
    x = x_ref[...]
    cb = cb_ref[...]

    @pl.when(pl.program_id(0) == 0)
    def _():
        csq_ref[...] = jnp.sum(cb * cb, axis=1, keepdims=True).reshape(1, _K)
        loss_ref[0, 0] = 0.0

    cross = jax.lax.dot_general(
        x, cb, (((1,), (1,)), ((), ())), preferred_element_type=jnp.float32
    )
    x_sq = jnp.sum(x * x, axis=1, keepdims=True)
    c_sq = csq_ref[...]
    dist_sq = jnp.maximum(x_sq + c_sq - 2.0 * cross, 0.0)
    dists = jnp.sqrt(dist_sq)
    min_d = jnp.min(dists, axis=1, keepdims=True)
    iota_f = jax.lax.broadcasted_iota(jnp.int32, dists.shape, 1).astype(jnp.float32)
    idx_f = jnp.min(
        jnp.where(dists == min_d, iota_f, float(_K)), axis=1, keepdims=True
    )
    idx_ref[...] = idx_f.astype(jnp.int32).reshape(1, 1, _BLK)
    onehot = jnp.where(iota_f == idx_f, 1.0, 0.0).astype(jnp.bfloat16)
    gathered = jax.lax.dot_general(
        onehot, cb.astype(jnp.bfloat16), (((1,), (0,)), ((), ())),
        preferred_element_type=jnp.float32,
    )
    quant_ref[...] = gathered
    loss_ref[0, 0] += jnp.sum(min_d * min_d)


def kernel(x, codebook):
    B, T, D = x.shape
    N = B * T
    xf = x.reshape(N, D)
    grid = N // _BLK
    quant, idx, loss_sum = pl.pallas_call(
        _vq_body,
        grid=(grid,),
        in_specs=[
            pl.BlockSpec((_BLK, D), lambda i: (i, 0)),
            pl.BlockSpec((_K, D), lambda i: (0, 0)),
        ],
        out_specs=[
            pl.BlockSpec((_BLK, D), lambda i: (i, 0)),
            pl.BlockSpec((1, 1, _BLK), lambda i: (i, 0, 0)),
            pl.BlockSpec(memory_space=pltpu.SMEM),
        ],
        out_shape=[
            jax.ShapeDtypeStruct((N, D), jnp.float32),
            jax.ShapeDtypeStruct((grid, 1, _BLK), jnp.int32),
            jax.ShapeDtypeStruct((1, 1), jnp.float32),
        ],
        scratch_shapes=[pltpu.VMEM((1, _K), jnp.float32)],
    )(xf, codebook)
    quantized = quant.reshape(B, T, D)
    indices = idx.reshape(B, T)
    quantize_loss = (2.0 / N / D) * loss_sum[0, 0]
    return (quantized, indices, quantize_loss)
